# baseline (speedup 1.0000x reference)
import functools

import jax
import jax.numpy as jnp
from jax import lax
from jax.experimental import pallas as pl
from jax.experimental.pallas import tpu as pltpu
from jax.experimental.pallas import tpu_sc as plsc

CHUNK = 128
NUM_CORES = 2
NUM_SUBCORES = 16
NW = NUM_CORES * NUM_SUBCORES
N_PAD = 10240
N_PAD_DEG = 16384
RPS = N_PAD // NUM_SUBCORES
_SC_PARAMS = pltpu.CompilerParams(use_tc_tiling_on_sc=False,
                                  needs_layout_passes=False)


def _rsqrt16(x):
    i = plsc.bitcast(x, jnp.int32)
    i = jnp.int32(0x5F3759DF) - (i >> 1)
    y = plsc.bitcast(i, jnp.float32)
    for _ in range(3):
        y = y * (1.5 - 0.5 * x * y * y)
    return y


def _fill_ones(ones_v):
    for i in range(CHUNK // 16):
        ones_v[pl.ds(i * 16, 16)] = jnp.full((16,), 1.0, jnp.float32)


WAVE = 8
CPW0 = 32
CPW1 = 48
CPWMAX = max(CPW0, CPW1)


def _run_aggregation(table, src_v, dst_v, rows_v, acc, gsems, ssems, cpw):
    nwaves = cpw // WAVE
    wrows = WAVE * CHUNK

    def fire_gathers(w, b):
        def fire(j, carry):
            pltpu.make_async_copy(
                table.at[src_v.at[w * WAVE + j]],
                rows_v.at[b, pl.ds(j * CHUNK, CHUNK), :], gsems[b]).start()
            return carry
        lax.fori_loop(0, WAVE, fire, 0)

    def fire_scatters(w, b):
        def fire(j, carry):
            pltpu.make_async_copy(
                rows_v.at[b, pl.ds(j * CHUNK, CHUNK), :],
                acc.at[dst_v.at[w * WAVE + j]], ssems[b]).start(add=True)
            return carry
        lax.fori_loop(0, WAVE, fire, 0)

    def drain(sem, buf_side):
        def w1(j, carry):
            pltpu.make_async_copy(
                table.at[pl.ds(0, CHUNK), :],
                rows_v.at[buf_side, pl.ds(0, CHUNK), :], sem).wait()
            return carry
        lax.fori_loop(0, WAVE, w1, 0)

    fire_gathers(0, 0)
    for w in range(nwaves):
        b = w & 1
        if w + 1 < nwaves:
            if w >= 1:
                drain(ssems[1 - b], 1 - b)
            fire_gathers(w + 1, 1 - b)
        drain(gsems[b], b)
        fire_scatters(w, b)
    drain(ssems[(nwaves - 1) & 1], (nwaves - 1) & 1)
    if nwaves >= 2:
        drain(ssems[(nwaves - 2) & 1], (nwaves - 2) & 1)


def _load_agg_idx(src_hbm, dst_hbm, src_v, dst_v, c, s):
    @pl.when(c == 0)
    def _():
        pltpu.sync_copy(src_hbm.at[pl.ds(s * CPW0, CPW0)],
                        src_v.at[pl.ds(0, CPW0)])
        pltpu.sync_copy(dst_hbm.at[pl.ds(s * CPW0, CPW0)],
                        dst_v.at[pl.ds(0, CPW0)])

    @pl.when(c == 1)
    def _():
        base = NUM_SUBCORES * CPW0
        pltpu.sync_copy(src_hbm.at[pl.ds(base + s * CPW1, CPW1)],
                        src_v.at[pl.ds(0, CPW1)])
        pltpu.sync_copy(dst_hbm.at[pl.ds(base + s * CPW1, CPW1)],
                        dst_v.at[pl.ds(0, CPW1)])


def _agg_both(table, src_v, dst_v, rows_v, acc, gsems, ssems, c):
    @pl.when(c == 0)
    def _():
        _run_aggregation(table, src_v, dst_v, rows_v, acc, gsems, ssems, CPW0)

    @pl.when(c == 1)
    def _():
        _run_aggregation(table, src_v, dst_v, rows_v, acc, gsems, ssems, CPW1)


def _sc_mega1(hw1, src2d, dst2d, z16, zdeg):
    nch = src2d.shape[0]
    assert nch == NUM_SUBCORES * (CPW0 + CPW1)
    dpw = nch // NUM_SUBCORES
    hid = hw1.shape[1]
    drps = N_PAD_DEG // NUM_SUBCORES
    mesh = plsc.VectorSubcoreMesh(core_axis_name="c", subcore_axis_name="s")

    @functools.partial(
        pl.kernel,
        mesh=mesh,
        out_type=(
            jax.ShapeDtypeStruct((NUM_CORES, N_PAD, hid), jnp.float32),
            jax.ShapeDtypeStruct((NUM_CORES, N_PAD, hid), jnp.float32),
            jax.ShapeDtypeStruct((N_PAD,), jnp.float32),
        ),
        compiler_params=_SC_PARAMS,
        scratch_types=[
            pltpu.VMEM((dpw, CHUNK), jnp.int32),
            pltpu.VMEM((CPWMAX, CHUNK), jnp.int32),
            pltpu.VMEM((CPWMAX, CHUNK), jnp.int32),
            pltpu.VMEM((CHUNK,), jnp.float32),
            pltpu.VMEM((RPS, 16), jnp.float32),
            pltpu.VMEM((RPS,), jnp.float32),
            pltpu.VMEM((RPS,), jnp.float32),
            pltpu.VMEM((2, WAVE * CHUNK, 16), jnp.float32),
            pltpu.VMEM_SHARED((N_PAD, 16), jnp.float32),
            pltpu.VMEM_SHARED((N_PAD_DEG,), jnp.float32),
            pltpu.SemaphoreType.DMA,
            pltpu.SemaphoreType.DMA,
            pltpu.SemaphoreType.DMA,
            pltpu.SemaphoreType.DMA,
            pltpu.SemaphoreType.DMA,
        ],
    )
    def mega1(hw_hbm, src_hbm, dst_hbm, z16_hbm, zd_hbm,
              p_hbm, hs1_hbm, dinv_hbm,
              degidx_v, src_v, dst_v, ones_v, hw_v, deg_v, dinv_v, rows_v,
              acc, dacc, dsem, gsem0, gsem1, ssem0, ssem1):
        c = lax.axis_index("c")
        s = lax.axis_index("s")
        pltpu.sync_copy(z16_hbm.at[pl.ds(s * RPS, RPS), :],
                        acc.at[pl.ds(s * RPS, RPS), :])
        pltpu.sync_copy(zd_hbm.at[pl.ds(s * drps, drps)],
                        dacc.at[pl.ds(s * drps, drps)])
        _fill_ones(ones_v)
        pltpu.sync_copy(dst_hbm.at[pl.ds(s * dpw, dpw)], degidx_v)
        plsc.subcore_barrier()

        def fire_deg(j, carry):
            pltpu.make_async_copy(ones_v, dacc.at[degidx_v.at[j]],
                                  dsem).start(add=True)
            return carry

        lax.fori_loop(0, dpw, fire_deg, 0)
        pltpu.sync_copy(hw_hbm.at[pl.ds(s * RPS, RPS), :], hw_v)
        _load_agg_idx(src_hbm, dst_hbm, src_v, dst_v, c, s)

        def drain_deg(j, carry):
            pltpu.make_async_copy(ones_v, dacc.at[pl.ds(0, CHUNK)],
                                  dsem).wait()
            return carry

        lax.fori_loop(0, dpw, drain_deg, 0)
        plsc.subcore_barrier()

        pltpu.sync_copy(dacc.at[pl.ds(s * RPS, RPS)], deg_v)

        def dinv_blk(b, carry):
            dinv_v[pl.ds(b * 16, 16)] = _rsqrt16(
                deg_v[pl.ds(b * 16, 16)] + 1.0)
            return carry

        lax.fori_loop(0, RPS // 16, dinv_blk, 0)

        def scale_row(r, carry):
            bidx = jnp.zeros((16,), jnp.int32) + r
            db = plsc.load_gather(dinv_v, [bidx])
            hw_v[r, :] = hw_v[r, :] * db
            return carry

        lax.fori_loop(0, RPS, scale_row, 0)
        pltpu.sync_copy(hw_v, hs1_hbm.at[c, pl.ds(s * RPS, RPS), :])

        @pl.when(c == 0)
        def _():
            pltpu.sync_copy(dinv_v, dinv_hbm.at[pl.ds(s * RPS, RPS)])

        plsc.subcore_barrier()

        _agg_both(hs1_hbm.at[c], src_v, dst_v, rows_v, acc,
                  (gsem0, gsem1), (ssem0, ssem1), c)
        plsc.subcore_barrier()
        pltpu.sync_copy(acc.at[pl.ds(s * RPS, RPS), :],
                        p_hbm.at[c, pl.ds(s * RPS, RPS), :])

    return mega1(hw1, src2d, dst2d, z16, zdeg)


def _sc_mega2(p, hw1, dinv_flat, b1, src2d, dst2d, z16):
    nch = src2d.shape[0]
    assert nch == NUM_SUBCORES * (CPW0 + CPW1)
    hid = hw1.shape[1]
    mesh = plsc.VectorSubcoreMesh(core_axis_name="c", subcore_axis_name="s")

    @functools.partial(
        pl.kernel,
        mesh=mesh,
        out_type=(
            jax.ShapeDtypeStruct((NUM_CORES, N_PAD, hid), jnp.float32),
            jax.ShapeDtypeStruct((NUM_CORES, N_PAD, hid), jnp.float32),
        ),
        compiler_params=_SC_PARAMS,
        scratch_types=[
            pltpu.VMEM((CPWMAX, CHUNK), jnp.int32),
            pltpu.VMEM((CPWMAX, CHUNK), jnp.int32),
            pltpu.VMEM((RPS, 16), jnp.float32),
            pltpu.VMEM((RPS, 16), jnp.float32),
            pltpu.VMEM((RPS,), jnp.float32),
            pltpu.VMEM((16,), jnp.float32),
            pltpu.VMEM((2, WAVE * CHUNK, 16), jnp.float32),
            pltpu.VMEM_SHARED((N_PAD, 16), jnp.float32),
            pltpu.SemaphoreType.DMA,
            pltpu.SemaphoreType.DMA,
            pltpu.SemaphoreType.DMA,
            pltpu.SemaphoreType.DMA,
        ],
    )
    def mega2(p_hbm, hw_hbm, dinv_hbm, b1_hbm, src_hbm, dst_hbm, z16_hbm,
              q_hbm, hs2_hbm,
              src_v, dst_v, p0_v, hw_v, dinv_v, b1_v, rows_v,
              acc, gsem0, gsem1, ssem0, ssem1):
        c = lax.axis_index("c")
        s = lax.axis_index("s")
        p1_v = rows_v.at[0, pl.ds(0, RPS), :]
        pltpu.sync_copy(z16_hbm.at[pl.ds(s * RPS, RPS), :],
                        acc.at[pl.ds(s * RPS, RPS), :])
        pltpu.sync_copy(p_hbm.at[0, pl.ds(s * RPS, RPS), :], p0_v)
        pltpu.sync_copy(p_hbm.at[1, pl.ds(s * RPS, RPS), :], p1_v)
        pltpu.sync_copy(hw_hbm.at[pl.ds(s * RPS, RPS), :], hw_v)
        pltpu.sync_copy(dinv_hbm.at[pl.ds(s * RPS, RPS)], dinv_v)
        pltpu.sync_copy(b1_hbm, b1_v)
        _load_agg_idx(src_hbm, dst_hbm, src_v, dst_v, c, s)
        b1v = b1_v[...]

        def row(r, carry):
            bidx = jnp.zeros((16,), jnp.int32) + r
            db = plsc.load_gather(dinv_v, [bidx])
            t = db * (p0_v[r, :] + rows_v[0, r, :] + db * hw_v[r, :]) + b1v
            hw_v[r, :] = jnp.maximum(t, 0.0) * db
            return carry

        lax.fori_loop(0, RPS, row, 0)
        pltpu.sync_copy(hw_v, hs2_hbm.at[c, pl.ds(s * RPS, RPS), :])
        plsc.subcore_barrier()

        _agg_both(hs2_hbm.at[c], src_v, dst_v, rows_v, acc,
                  (gsem0, gsem1), (ssem0, ssem1), c)
        plsc.subcore_barrier()
        pltpu.sync_copy(acc.at[pl.ds(s * RPS, RPS), :],
                        q_hbm.at[c, pl.ds(s * RPS, RPS), :])

    return mega2(p, hw1, dinv_flat, b1, src2d, dst2d, z16)


def _tc_mm1(x, w1, block_n=2000):
    n, f_in = x.shape
    hid = w1.shape[1]
    grid = n // block_n

    def body(x_ref, w_ref, out_ref):
        out_ref[...] = jnp.dot(x_ref[...], w_ref[...],
                               preferred_element_type=jnp.float32)

    return pl.pallas_call(
        body,
        grid=(grid,),
        in_specs=[
            pl.BlockSpec((block_n, f_in), lambda i: (i, 0)),
            pl.BlockSpec((f_in, hid), lambda i: (0, 0)),
        ],
        out_specs=pl.BlockSpec((block_n, hid), lambda i: (i, 0)),
        out_shape=jax.ShapeDtypeStruct((N_PAD, hid), jnp.float32),
    )(x, w1)


def _tc_final(q0, q1, hs2, dinv, w2, b2, block_n=2000):
    n, hid = hs2.shape
    c_out = w2.shape[1]
    grid = n // block_n

    def body(q0_ref, q1_ref, hs_ref, dinv_ref, w2_ref, b2_ref, out_ref):
        t = dinv_ref[...] * (q0_ref[...] + q1_ref[...] + hs_ref[...])
        logits = jnp.dot(t, w2_ref[...],
                         preferred_element_type=jnp.float32) + b2_ref[...]
        m = jnp.max(logits, axis=1, keepdims=True)
        lse = jnp.log(jnp.sum(jnp.exp(logits - m), axis=1, keepdims=True)) + m
        out_ref[...] = logits - lse

    return pl.pallas_call(
        body,
        grid=(grid,),
        in_specs=[
            pl.BlockSpec((block_n, hid), lambda i: (i, 0)),
            pl.BlockSpec((block_n, hid), lambda i: (i, 0)),
            pl.BlockSpec((block_n, hid), lambda i: (i, 0)),
            pl.BlockSpec((block_n, 1), lambda i: (i, 0)),
            pl.BlockSpec((hid, c_out), lambda i: (0, 0)),
            pl.BlockSpec((1, c_out), lambda i: (0, 0)),
        ],
        out_specs=pl.BlockSpec((block_n, c_out), lambda i: (i, 0)),
        out_shape=jax.ShapeDtypeStruct((n, c_out), jnp.float32),
    )(q0, q1, hs2, dinv, w2, b2)


def kernel(x, edge_index, W1, b1, W2, b2):
    n, f_in = x.shape
    e = edge_index.shape[1]
    hid = W1.shape[1]

    e_pad = ((e + CHUNK * NW - 1) // (CHUNK * NW)) * (CHUNK * NW)
    pad = e_pad - e
    src_full = jnp.concatenate([edge_index[0], jnp.zeros((pad,), jnp.int32)])
    dst_full = jnp.concatenate(
        [edge_index[1], jnp.full((pad,), n, jnp.int32)])
    src2d = src_full.reshape(e_pad // CHUNK, CHUNK)
    dst2d = dst_full.reshape(e_pad // CHUNK, CHUNK)

    z16 = jnp.zeros((N_PAD, hid), jnp.float32)
    zdeg = jnp.zeros((N_PAD_DEG,), jnp.float32)

    hw1 = _tc_mm1(x, W1)
    p, hs1x2, dinv_flat = _sc_mega1(hw1, src2d, dst2d, z16, zdeg)
    del hs1x2
    q, hs2x2 = _sc_mega2(p, hw1, dinv_flat, b1, src2d, dst2d, z16)
    dinv = dinv_flat[:n].reshape(n, 1)
    return _tc_final(q[0, :n], q[1, :n], hs2x2[0, :n], dinv, W2,
                     b2.reshape(1, W2.shape[1]))


# speedup vs baseline: 1.0222x; 1.0222x over previous
import functools

import jax
import jax.numpy as jnp
from jax import lax
from jax.experimental import pallas as pl
from jax.experimental.pallas import tpu as pltpu
from jax.experimental.pallas import tpu_sc as plsc

CHUNK = 128
NUM_CORES = 2
NUM_SUBCORES = 16
NW = NUM_CORES * NUM_SUBCORES
N_PAD = 10240
N_PAD_DEG = 16384
RPS = N_PAD // NUM_SUBCORES
_SC_PARAMS = pltpu.CompilerParams(use_tc_tiling_on_sc=False,
                                  needs_layout_passes=False)


def _rsqrt16(x):
    i = plsc.bitcast(x, jnp.int32)
    i = jnp.int32(0x5F3759DF) - (i >> 1)
    y = plsc.bitcast(i, jnp.float32)
    for _ in range(3):
        y = y * (1.5 - 0.5 * x * y * y)
    return y


def _fill_ones(ones_v):
    for i in range(CHUNK // 16):
        ones_v[pl.ds(i * 16, 16)] = jnp.full((16,), 1.0, jnp.float32)


WAVE = 8
CPW0 = 40
CPW1 = 40
CPWMAX = max(CPW0, CPW1)


def _run_aggregation(table, src_v, dst_v, rows_v, acc, gsems, ssems, cpw):
    nwaves = cpw // WAVE
    wrows = WAVE * CHUNK

    def fire_gathers(w, b):
        def fire(j, carry):
            pltpu.make_async_copy(
                table.at[src_v.at[w * WAVE + j]],
                rows_v.at[b, pl.ds(j * CHUNK, CHUNK), :], gsems[b]).start()
            return carry
        lax.fori_loop(0, WAVE, fire, 0)

    def fire_scatters(w, b):
        def fire(j, carry):
            pltpu.make_async_copy(
                rows_v.at[b, pl.ds(j * CHUNK, CHUNK), :],
                acc.at[dst_v.at[w * WAVE + j]], ssems[b]).start(add=True)
            return carry
        lax.fori_loop(0, WAVE, fire, 0)

    def drain(sem, buf_side):
        def w1(j, carry):
            pltpu.make_async_copy(
                table.at[pl.ds(0, CHUNK), :],
                rows_v.at[buf_side, pl.ds(0, CHUNK), :], sem).wait()
            return carry
        lax.fori_loop(0, WAVE, w1, 0)

    fire_gathers(0, 0)
    for w in range(nwaves):
        b = w & 1
        if w + 1 < nwaves:
            if w >= 1:
                drain(ssems[1 - b], 1 - b)
            fire_gathers(w + 1, 1 - b)
        drain(gsems[b], b)
        fire_scatters(w, b)
    drain(ssems[(nwaves - 1) & 1], (nwaves - 1) & 1)
    if nwaves >= 2:
        drain(ssems[(nwaves - 2) & 1], (nwaves - 2) & 1)


def _load_agg_idx(src_hbm, dst_hbm, src_v, dst_v, c, s):
    @pl.when(c == 0)
    def _():
        pltpu.sync_copy(src_hbm.at[pl.ds(s * CPW0, CPW0)],
                        src_v.at[pl.ds(0, CPW0)])
        pltpu.sync_copy(dst_hbm.at[pl.ds(s * CPW0, CPW0)],
                        dst_v.at[pl.ds(0, CPW0)])

    @pl.when(c == 1)
    def _():
        base = NUM_SUBCORES * CPW0
        pltpu.sync_copy(src_hbm.at[pl.ds(base + s * CPW1, CPW1)],
                        src_v.at[pl.ds(0, CPW1)])
        pltpu.sync_copy(dst_hbm.at[pl.ds(base + s * CPW1, CPW1)],
                        dst_v.at[pl.ds(0, CPW1)])


def _agg_both(table, src_v, dst_v, rows_v, acc, gsems, ssems, c):
    @pl.when(c == 0)
    def _():
        _run_aggregation(table, src_v, dst_v, rows_v, acc, gsems, ssems, CPW0)

    @pl.when(c == 1)
    def _():
        _run_aggregation(table, src_v, dst_v, rows_v, acc, gsems, ssems, CPW1)


def _sc_mega1(hw1, src2d, dst2d, z16, zdeg):
    nch = src2d.shape[0]
    assert nch == NUM_SUBCORES * (CPW0 + CPW1)
    dpw = nch // NUM_SUBCORES
    hid = hw1.shape[1]
    drps = N_PAD_DEG // NUM_SUBCORES
    mesh = plsc.VectorSubcoreMesh(core_axis_name="c", subcore_axis_name="s")

    @functools.partial(
        pl.kernel,
        mesh=mesh,
        out_type=(
            jax.ShapeDtypeStruct((NUM_CORES, N_PAD, hid), jnp.float32),
            jax.ShapeDtypeStruct((NUM_CORES, N_PAD, hid), jnp.float32),
            jax.ShapeDtypeStruct((N_PAD,), jnp.float32),
        ),
        compiler_params=_SC_PARAMS,
        scratch_types=[
            pltpu.VMEM((dpw, CHUNK), jnp.int32),
            pltpu.VMEM((CPWMAX, CHUNK), jnp.int32),
            pltpu.VMEM((CPWMAX, CHUNK), jnp.int32),
            pltpu.VMEM((CHUNK,), jnp.float32),
            pltpu.VMEM((RPS, 16), jnp.float32),
            pltpu.VMEM((RPS,), jnp.float32),
            pltpu.VMEM((RPS,), jnp.float32),
            pltpu.VMEM((2, WAVE * CHUNK, 16), jnp.float32),
            pltpu.VMEM_SHARED((N_PAD, 16), jnp.float32),
            pltpu.VMEM_SHARED((N_PAD_DEG,), jnp.float32),
            pltpu.SemaphoreType.DMA,
            pltpu.SemaphoreType.DMA,
            pltpu.SemaphoreType.DMA,
            pltpu.SemaphoreType.DMA,
            pltpu.SemaphoreType.DMA,
        ],
    )
    def mega1(hw_hbm, src_hbm, dst_hbm, z16_hbm, zd_hbm,
              p_hbm, hs1_hbm, dinv_hbm,
              degidx_v, src_v, dst_v, ones_v, hw_v, deg_v, dinv_v, rows_v,
              acc, dacc, dsem, gsem0, gsem1, ssem0, ssem1):
        c = lax.axis_index("c")
        s = lax.axis_index("s")
        pltpu.sync_copy(z16_hbm.at[pl.ds(s * RPS, RPS), :],
                        acc.at[pl.ds(s * RPS, RPS), :])
        pltpu.sync_copy(zd_hbm.at[pl.ds(s * drps, drps)],
                        dacc.at[pl.ds(s * drps, drps)])
        _fill_ones(ones_v)
        pltpu.sync_copy(dst_hbm.at[pl.ds(s * dpw, dpw)], degidx_v)
        plsc.subcore_barrier()

        def fire_deg(j, carry):
            pltpu.make_async_copy(ones_v, dacc.at[degidx_v.at[j]],
                                  dsem).start(add=True)
            return carry

        lax.fori_loop(0, dpw, fire_deg, 0)
        pltpu.sync_copy(hw_hbm.at[pl.ds(s * RPS, RPS), :], hw_v)
        _load_agg_idx(src_hbm, dst_hbm, src_v, dst_v, c, s)

        def drain_deg(j, carry):
            pltpu.make_async_copy(ones_v, dacc.at[pl.ds(0, CHUNK)],
                                  dsem).wait()
            return carry

        lax.fori_loop(0, dpw, drain_deg, 0)
        plsc.subcore_barrier()

        pltpu.sync_copy(dacc.at[pl.ds(s * RPS, RPS)], deg_v)

        def dinv_blk(b, carry):
            dinv_v[pl.ds(b * 16, 16)] = _rsqrt16(
                deg_v[pl.ds(b * 16, 16)] + 1.0)
            return carry

        lax.fori_loop(0, RPS // 16, dinv_blk, 0)

        def scale_row(r, carry):
            bidx = jnp.zeros((16,), jnp.int32) + r
            db = plsc.load_gather(dinv_v, [bidx])
            hw_v[r, :] = hw_v[r, :] * db
            return carry

        lax.fori_loop(0, RPS, scale_row, 0)
        pltpu.sync_copy(hw_v, hs1_hbm.at[c, pl.ds(s * RPS, RPS), :])

        @pl.when(c == 0)
        def _():
            pltpu.sync_copy(dinv_v, dinv_hbm.at[pl.ds(s * RPS, RPS)])

        plsc.subcore_barrier()

        _agg_both(hs1_hbm.at[c], src_v, dst_v, rows_v, acc,
                  (gsem0, gsem1), (ssem0, ssem1), c)
        plsc.subcore_barrier()
        pltpu.sync_copy(acc.at[pl.ds(s * RPS, RPS), :],
                        p_hbm.at[c, pl.ds(s * RPS, RPS), :])

    return mega1(hw1, src2d, dst2d, z16, zdeg)


def _sc_mega2(p, hw1, dinv_flat, b1, src2d, dst2d, z16):
    nch = src2d.shape[0]
    assert nch == NUM_SUBCORES * (CPW0 + CPW1)
    hid = hw1.shape[1]
    mesh = plsc.VectorSubcoreMesh(core_axis_name="c", subcore_axis_name="s")

    @functools.partial(
        pl.kernel,
        mesh=mesh,
        out_type=(
            jax.ShapeDtypeStruct((NUM_CORES, N_PAD, hid), jnp.float32),
            jax.ShapeDtypeStruct((NUM_CORES, N_PAD, hid), jnp.float32),
        ),
        compiler_params=_SC_PARAMS,
        scratch_types=[
            pltpu.VMEM((CPWMAX, CHUNK), jnp.int32),
            pltpu.VMEM((CPWMAX, CHUNK), jnp.int32),
            pltpu.VMEM((RPS, 16), jnp.float32),
            pltpu.VMEM((RPS, 16), jnp.float32),
            pltpu.VMEM((RPS,), jnp.float32),
            pltpu.VMEM((16,), jnp.float32),
            pltpu.VMEM((2, WAVE * CHUNK, 16), jnp.float32),
            pltpu.VMEM_SHARED((N_PAD, 16), jnp.float32),
            pltpu.SemaphoreType.DMA,
            pltpu.SemaphoreType.DMA,
            pltpu.SemaphoreType.DMA,
            pltpu.SemaphoreType.DMA,
        ],
    )
    def mega2(p_hbm, hw_hbm, dinv_hbm, b1_hbm, src_hbm, dst_hbm, z16_hbm,
              q_hbm, hs2_hbm,
              src_v, dst_v, p0_v, hw_v, dinv_v, b1_v, rows_v,
              acc, gsem0, gsem1, ssem0, ssem1):
        c = lax.axis_index("c")
        s = lax.axis_index("s")
        p1_v = rows_v.at[0, pl.ds(0, RPS), :]
        pltpu.sync_copy(z16_hbm.at[pl.ds(s * RPS, RPS), :],
                        acc.at[pl.ds(s * RPS, RPS), :])
        pltpu.sync_copy(p_hbm.at[0, pl.ds(s * RPS, RPS), :], p0_v)
        pltpu.sync_copy(p_hbm.at[1, pl.ds(s * RPS, RPS), :], p1_v)
        pltpu.sync_copy(hw_hbm.at[pl.ds(s * RPS, RPS), :], hw_v)
        pltpu.sync_copy(dinv_hbm.at[pl.ds(s * RPS, RPS)], dinv_v)
        pltpu.sync_copy(b1_hbm, b1_v)
        _load_agg_idx(src_hbm, dst_hbm, src_v, dst_v, c, s)
        b1v = b1_v[...]

        def row(r, carry):
            bidx = jnp.zeros((16,), jnp.int32) + r
            db = plsc.load_gather(dinv_v, [bidx])
            t = db * (p0_v[r, :] + rows_v[0, r, :] + db * hw_v[r, :]) + b1v
            hw_v[r, :] = jnp.maximum(t, 0.0) * db
            return carry

        lax.fori_loop(0, RPS, row, 0)
        pltpu.sync_copy(hw_v, hs2_hbm.at[c, pl.ds(s * RPS, RPS), :])
        plsc.subcore_barrier()

        _agg_both(hs2_hbm.at[c], src_v, dst_v, rows_v, acc,
                  (gsem0, gsem1), (ssem0, ssem1), c)
        plsc.subcore_barrier()
        pltpu.sync_copy(acc.at[pl.ds(s * RPS, RPS), :],
                        q_hbm.at[c, pl.ds(s * RPS, RPS), :])

    return mega2(p, hw1, dinv_flat, b1, src2d, dst2d, z16)


def _tc_mm1(x, w1, block_n=2000):
    n, f_in = x.shape
    hid = w1.shape[1]
    grid = n // block_n

    def body(x_ref, w_ref, out_ref):
        out_ref[...] = jnp.dot(x_ref[...], w_ref[...],
                               preferred_element_type=jnp.float32)

    return pl.pallas_call(
        body,
        grid=(grid,),
        in_specs=[
            pl.BlockSpec((block_n, f_in), lambda i: (i, 0)),
            pl.BlockSpec((f_in, hid), lambda i: (0, 0)),
        ],
        out_specs=pl.BlockSpec((block_n, hid), lambda i: (i, 0)),
        out_shape=jax.ShapeDtypeStruct((N_PAD, hid), jnp.float32),
    )(x, w1)


def _tc_final(q0, q1, hs2, dinv, w2, b2, block_n=2000):
    n, hid = hs2.shape
    c_out = w2.shape[1]
    grid = n // block_n

    def body(q0_ref, q1_ref, hs_ref, dinv_ref, w2_ref, b2_ref, out_ref):
        t = dinv_ref[...] * (q0_ref[...] + q1_ref[...] + hs_ref[...])
        logits = jnp.dot(t, w2_ref[...],
                         preferred_element_type=jnp.float32) + b2_ref[...]
        m = jnp.max(logits, axis=1, keepdims=True)
        lse = jnp.log(jnp.sum(jnp.exp(logits - m), axis=1, keepdims=True)) + m
        out_ref[...] = logits - lse

    return pl.pallas_call(
        body,
        grid=(grid,),
        in_specs=[
            pl.BlockSpec((block_n, hid), lambda i: (i, 0)),
            pl.BlockSpec((block_n, hid), lambda i: (i, 0)),
            pl.BlockSpec((block_n, hid), lambda i: (i, 0)),
            pl.BlockSpec((block_n, 1), lambda i: (i, 0)),
            pl.BlockSpec((hid, c_out), lambda i: (0, 0)),
            pl.BlockSpec((1, c_out), lambda i: (0, 0)),
        ],
        out_specs=pl.BlockSpec((block_n, c_out), lambda i: (i, 0)),
        out_shape=jax.ShapeDtypeStruct((n, c_out), jnp.float32),
    )(q0, q1, hs2, dinv, w2, b2)


def kernel(x, edge_index, W1, b1, W2, b2):
    n, f_in = x.shape
    e = edge_index.shape[1]
    hid = W1.shape[1]

    e_pad = ((e + CHUNK * NW - 1) // (CHUNK * NW)) * (CHUNK * NW)
    pad = e_pad - e
    src_full = jnp.concatenate([edge_index[0], jnp.zeros((pad,), jnp.int32)])
    dst_full = jnp.concatenate(
        [edge_index[1], jnp.full((pad,), n, jnp.int32)])
    src2d = src_full.reshape(e_pad // CHUNK, CHUNK)
    dst2d = dst_full.reshape(e_pad // CHUNK, CHUNK)

    z16 = jnp.zeros((N_PAD, hid), jnp.float32)
    zdeg = jnp.zeros((N_PAD_DEG,), jnp.float32)

    hw1 = _tc_mm1(x, W1)
    p, hs1x2, dinv_flat = _sc_mega1(hw1, src2d, dst2d, z16, zdeg)
    del hs1x2
    q, hs2x2 = _sc_mega2(p, hw1, dinv_flat, b1, src2d, dst2d, z16)
    dinv = dinv_flat[:n].reshape(n, 1)
    return _tc_final(q[0, :n], q[1, :n], hs2x2[0, :n], dinv, W2,
                     b2.reshape(1, W2.shape[1]))
